# baseline (speedup 1.0000x reference)
import functools

import jax
import jax.numpy as jnp
from jax import lax
from jax.experimental import pallas as pl
from jax.experimental.pallas import tpu as pltpu
from jax.experimental.pallas import tpu_sc as plsc

_SC_CORES = 2
_SC_SUBCORES = 16
_SC_WORKERS = _SC_CORES * _SC_SUBCORES

_BLK = 32


def _make_sc_row_gather(T_rows, D, R):
    per_w = T_rows // _SC_WORKERS
    n_chunks = per_w // R
    mesh = plsc.VectorSubcoreMesh(core_axis_name="c", subcore_axis_name="s")

    @functools.partial(
        pl.kernel,
        out_type=jax.ShapeDtypeStruct((T_rows, D), jnp.float32),
        mesh=mesh,
        scratch_types=[
            pltpu.VMEM((R,), jnp.int32),
            pltpu.VMEM((R, D), jnp.float32),
            pltpu.SemaphoreType.DMA,
        ],
    )
    def gather_kernel(src_hbm, idx_hbm, out_hbm, idx_v, rows_v, sem):
        wid = lax.axis_index("s") * _SC_CORES + lax.axis_index("c")
        for c in range(n_chunks):
            base = wid * per_w + c * R
            pltpu.sync_copy(idx_hbm.at[pl.ds(base, R)], idx_v)
            pltpu.async_copy(src_hbm.at[idx_v], rows_v, sem).wait()
            pltpu.sync_copy(rows_v, out_hbm.at[pl.ds(base, R)])

    return gather_kernel


def _ragged_matmul_body(off_r, ord_r, wsc_r, xs_ref, w_ref, o_ref):
    e = pl.program_id(0)
    start = off_r[e]
    end = off_r[e + 1]
    blk0 = start // _BLK
    n_chunks = (end + _BLK - 1) // _BLK - blk0

    def chunk(j, carry):
        s0 = (blk0 + j) * _BLK
        y = jnp.dot(
            xs_ref[pl.ds(s0, _BLK), :], w_ref[0],
            preferred_element_type=jnp.float32,
        )
        for i in range(_BLK):
            g = s0 + i

            @pl.when((g >= start) & (g < end))
            def _():
                t = ord_r[g]
                o_ref[pl.ds(t, 1), :] = y[i : i + 1, :] * wsc_r[t]

        return carry

    lax.fori_loop(0, n_chunks, chunk, 0)


def _ragged_matmul(xs, W, offsets_ext, order, wvec):
    T, K = xs.shape
    E, _, H = W.shape
    grid_spec = pltpu.PrefetchScalarGridSpec(
        num_scalar_prefetch=3,
        grid=(E,),
        in_specs=[
            pl.BlockSpec((T, K), lambda e, off, ordr, wsc: (0, 0)),
            pl.BlockSpec((1, K, H), lambda e, off, ordr, wsc: (e, 0, 0)),
        ],
        out_specs=pl.BlockSpec((T, H), lambda e, off, ordr, wsc: (0, 0)),
    )
    return pl.pallas_call(
        _ragged_matmul_body,
        grid_spec=grid_spec,
        out_shape=jax.ShapeDtypeStruct((T, H), jnp.float32),
    )(offsets_ext, order, wvec, xs, W)


def kernel(intermediate_states, down_weight, full_topk_ids, full_topk_weight):
    x = intermediate_states
    W = down_weight
    T, K = x.shape
    E, _, H = W.shape

    flat_ids = full_topk_ids.reshape(T).astype(jnp.int32)
    order = jnp.argsort(flat_ids).astype(jnp.int32)
    cmp = flat_ids[None, :] < jnp.arange(1, E + 1, dtype=jnp.int32)[:, None]
    offsets_ext = jnp.concatenate(
        [jnp.zeros((1,), jnp.int32), cmp.sum(axis=1).astype(jnp.int32)]
    )
    wvec = full_topk_weight.astype(jnp.float32).reshape(T)
    xs = _make_sc_row_gather(T, K, 64)(x, order)
    out = _ragged_matmul(xs, W, offsets_ext, order, wvec)
    return out
